# baseline (speedup 1.0000x reference)
import jax
import jax.numpy as jnp
from jax.experimental import pallas as pl
from jax.experimental.pallas import tpu as pltpu

_GAMMA = 2.0
_EPS = 1e-07

_BR = 4096


def _focal_body(x_ref, tgt_ref, out_ref):
    x = x_ref[...]
    tgt = tgt_ref[...]
    br, c = x.shape

    m = jnp.max(x, axis=1, keepdims=True)
    e = jnp.exp(x - m)
    s = jnp.sum(e, axis=1, keepdims=True)

    col = jax.lax.broadcasted_iota(jnp.int32, (br, c), 1)
    onehot = col == tgt.reshape(br, 1)
    et = jnp.sum(jnp.where(onehot, e, 0.0), axis=1, keepdims=True)

    p = et / s
    p = jnp.clip(p, _EPS, 1.0 - _EPS)
    one_m_p = 1.0 - p
    loss = -jnp.log(p) * one_m_p * one_m_p

    @pl.when(pl.program_id(0) == 0)
    def _():
        out_ref[0, 0] = 0.0

    out_ref[0, 0] += jnp.sum(loss)


@jax.jit
def _focal_loss(inp, tgt):
    b, c = inp.shape
    grid = b // _BR
    out = pl.pallas_call(
        _focal_body,
        grid=(grid,),
        in_specs=[
            pl.BlockSpec((_BR, c), lambda i: (i, 0)),
            pl.BlockSpec((1, _BR), lambda i: (0, i)),
        ],
        out_specs=pl.BlockSpec(
            (1, 1), lambda i: (0, 0), memory_space=pltpu.SMEM
        ),
        out_shape=jax.ShapeDtypeStruct((1, 1), jnp.float32),
    )(inp, tgt.reshape(1, b).astype(jnp.int32))
    return out[0, 0]


def kernel(input, target):
    return _focal_loss(input, target)


# speedup vs baseline: 3.0237x; 3.0237x over previous
import jax
import jax.numpy as jnp
from jax.experimental import pallas as pl
from jax.experimental.pallas import tpu as pltpu

_GAMMA = 2.0
_EPS = 1e-07

_BC = 2048


def _focal_body(x_ref, tgt_ref, out_ref):
    x = x_ref[...]
    tgt = tgt_ref[...]
    c, bc = x.shape

    m = jnp.max(x, axis=0, keepdims=True)
    e = jnp.exp(x - m)
    s = jnp.sum(e, axis=0, keepdims=True)

    row = jax.lax.broadcasted_iota(jnp.int32, (c, bc), 0)
    onehot = row == tgt
    et = jnp.sum(jnp.where(onehot, e, 0.0), axis=0, keepdims=True)

    p = et / s
    p = jnp.clip(p, _EPS, 1.0 - _EPS)
    one_m_p = 1.0 - p
    loss = -jnp.log(p) * one_m_p * one_m_p

    @pl.when(pl.program_id(0) == 0)
    def _():
        out_ref[0, 0] = 0.0

    out_ref[0, 0] += jnp.sum(loss)


@jax.jit
def _focal_loss(inp, tgt):
    b, c = inp.shape
    xt = inp.T
    grid = b // _BC
    out = pl.pallas_call(
        _focal_body,
        grid=(grid,),
        in_specs=[
            pl.BlockSpec((c, _BC), lambda i: (0, i)),
            pl.BlockSpec((1, _BC), lambda i: (0, i)),
        ],
        out_specs=pl.BlockSpec(
            (1, 1), lambda i: (0, 0), memory_space=pltpu.SMEM
        ),
        out_shape=jax.ShapeDtypeStruct((1, 1), jnp.float32),
    )(xt, tgt.reshape(1, b).astype(jnp.int32))
    return out[0, 0]


def kernel(input, target):
    return _focal_loss(input, target)
